# baseline (speedup 1.0000x reference)
import functools

import jax
import jax.numpy as jnp
from jax import lax
from jax.experimental import pallas as pl
from jax.experimental.pallas import tpu as pltpu
from jax.experimental.pallas import tpu_sc as plsc

NC, NS, L = 2, 16, 16
NW = NC * NS
D = 128
B, TPH = 1024, 200
N = B * TPH
PW = N // NW
WI = 128
W = WI
NWIN = PW // W
NROW = PW // WI
NBUF = 3
NT, NB = 8, 6
CCH = D // L
VP = 1024
RPT = VP // NS
MROWS = NT * VP


def _make_kernel():
    mesh = plsc.VectorSubcoreMesh(core_axis_name="c", subcore_axis_name="s")

    scratch = (
        [pltpu.VMEM((NROW, WI), jnp.int32)] * 2
        + [pltpu.VMEM((NB, D), jnp.float32)]
        + [pltpu.VMEM((W, D), jnp.float32)] * NBUF
        + [pltpu.VMEM_SHARED((MROWS, D), jnp.float32)]
        + [pltpu.SemaphoreType.DMA] * (2 * NBUF)
    )

    @functools.partial(
        pl.kernel,
        out_type=jax.ShapeDtypeStruct((N, D), jnp.float32),
        mesh=mesh,
        compiler_params=pltpu.CompilerParams(needs_layout_passes=False),
        scratch_types=scratch,
    )
    def k(ph_ids_hbm, tone_ids_hbm, bnd_ids_hbm,
          ph_tab_hbm, tone_tab_hbm, bnd_tab_hbm,
          out_hbm,
          ids_v, bid_v, bnd_tab_v, *rest):
        rows = rest[:NBUF]
        tab_sp = rest[NBUF]
        gsems = rest[NBUF + 1:2 * NBUF + 1]
        osems = rest[2 * NBUF + 1:]

        wid = lax.axis_index("s") * NC + lax.axis_index("c")
        base = wid * PW

        pltpu.sync_copy(ph_ids_hbm.at[wid], ids_v)
        pltpu.sync_copy(tone_ids_hbm.at[wid], bid_v)
        def build_fid(i, carry):
            r = i // (WI // L)
            kk = i - r * (WI // L)
            p = ids_v[r, pl.ds(kk * L, L)]
            t = bid_v[r, pl.ds(kk * L, L)]
            ids_v[r, pl.ds(kk * L, L)] = t * VP + p
            return carry
        lax.fori_loop(0, NROW * (WI // L), build_fid, 0)
        pltpu.sync_copy(bnd_ids_hbm.at[wid], bid_v)
        pltpu.sync_copy(bnd_tab_hbm, bnd_tab_v)

        sid = lax.axis_index("s")
        pltpu.sync_copy(ph_tab_hbm.at[pl.ds(sid * RPT, RPT)],
                        rows[0].at[pl.ds(0, RPT)])
        pltpu.sync_copy(tone_tab_hbm, rows[1].at[pl.ds(0, NT)])
        for t in range(NT):
            dt = []
            for c in range(CCH):
                v = rows[1][t, pl.ds(c * L, L)]
                if t > 0:
                    v = v - rows[1][t - 1, pl.ds(c * L, L)]
                dt.append(v)

            def add_dt(r, carry):
                for c in range(CCH):
                    plsc.addupdate(rows[0].at[r, pl.ds(c * L, L)], dt[c])
                return carry
            lax.fori_loop(0, RPT, add_dt, 0)
            pltpu.sync_copy(rows[0].at[pl.ds(0, RPT)],
                            tab_sp.at[pl.ds(t * VP + sid * RPT, RPT)])
        plsc.subcore_barrier()

        def g_start(w, p):
            pltpu.async_copy(tab_sp.at[ids_v.at[w]], rows[p], gsems[p])

        def g_wait(w, p):
            pltpu.make_async_copy(tab_sp.at[ids_v.at[w]], rows[p],
                                  gsems[p]).wait()

        def o_copy(w, p):
            return pltpu.make_async_copy(
                rows[p], out_hbm.at[pl.ds(base + w * W, W)], osems[p])

        def compute(w, p):
            def chunk(ck, carry2):
                pos0 = ck * L
                bvec = bid_v[w, pl.ds(pos0, L)]
                for j in range(L):
                    b = bvec[j]
                    for c in range(CCH):
                        val = bnd_tab_v[b, pl.ds(c * L, L)]
                        plsc.addupdate(rows[p].at[pos0 + j, pl.ds(c * L, L)],
                                       val)
                return carry2
            lax.fori_loop(0, W // L, chunk, 0)

        def step(w, par, do_owait, do_gstart):
            g_wait(w, par)
            if do_owait:
                o_copy(w - 2, (par + 1) % NBUF).wait()
            if do_gstart:
                g_start(w + 1, (par + 1) % NBUF)
            compute(w, par)
            o_copy(w, par).start()

        g_start(0, 0)

        for par in range(NBUF):
            step(par, par, par >= 2, True)

        def round_body(r, carry):
            w0 = r * NBUF
            for par in range(NBUF):
                step(w0 + par, par, True, True)
            return carry
        lax.fori_loop(1, (NWIN - 1) // NBUF, round_body, 0)

        for w in range(((NWIN - 1) // NBUF) * NBUF, NWIN):
            step(w, w % NBUF, True, w + 1 < NWIN)

        for w in (NWIN - 2, NWIN - 1):
            o_copy(w, w % NBUF).wait()

    return k


_kernel_fn = _make_kernel()


@jax.jit
def _run(ph_ids, tone_ids, boundary_ids, ph_table, tone_table, boundary_table):
    ph = ph_ids.reshape(NW, NROW, WI).astype(jnp.int32)
    tn = tone_ids.reshape(NW, NROW, WI).astype(jnp.int32)
    bd = boundary_ids.reshape(NW, NROW, WI).astype(jnp.int32)
    ph_table = jnp.concatenate(
        [ph_table, jnp.zeros((VP - ph_table.shape[0], D), ph_table.dtype)])
    out = _kernel_fn(ph, tn, bd, ph_table, tone_table, boundary_table)
    return out.reshape(B, TPH, D)


def kernel(ph_ids, tone_ids, boundary_ids, ph_table, tone_table, boundary_table):
    return _run(ph_ids, tone_ids, boundary_ids, ph_table, tone_table,
                boundary_table)


# speedup vs baseline: 1.6746x; 1.6746x over previous
import functools

import jax
import jax.numpy as jnp
from jax import lax
from jax.experimental import pallas as pl
from jax.experimental.pallas import tpu as pltpu
from jax.experimental.pallas import tpu_sc as plsc

NC, NS, L = 2, 16, 16
NW = NC * NS
D = 128
B, TPH = 1024, 200
N = B * TPH
PW = N // NW
WI = 128
NROW = PW // WI
W2 = 64
NWIN = PW // W2
NBUF = 3
NT, NB = 8, 6
CCH = D // L
VP = 1024
RPT = VP // NS
MROWS = NT * VP
TROWS = MROWS + 8


def _make_kernel():
    mesh = plsc.VectorSubcoreMesh(core_axis_name="c", subcore_axis_name="s")

    scratch = (
        [pltpu.VMEM((NROW, WI), jnp.int32)] * 2
        + [pltpu.VMEM((2 * W2, D), jnp.float32)] * NBUF
        + [pltpu.VMEM_SHARED((TROWS, D), jnp.float32)]
        + [pltpu.SemaphoreType.DMA] * (3 * NBUF)
    )

    @functools.partial(
        pl.kernel,
        out_type=jax.ShapeDtypeStruct((N, D), jnp.float32),
        mesh=mesh,
        compiler_params=pltpu.CompilerParams(needs_layout_passes=False),
        scratch_types=scratch,
    )
    def k(ph_ids_hbm, tone_ids_hbm, bnd_ids_hbm,
          ph_tab_hbm, tone_tab_hbm, bnd_tab_hbm,
          out_hbm,
          ids_v, bid_v, *rest):
        rows = rest[:NBUF]
        tab_sp = rest[NBUF]
        gasems = rest[NBUF + 1:2 * NBUF + 1]
        gbsems = rest[2 * NBUF + 1:3 * NBUF + 1]
        osems = rest[3 * NBUF + 1:]

        wid = lax.axis_index("s") * NC + lax.axis_index("c")
        base = wid * PW

        pltpu.sync_copy(ph_ids_hbm.at[wid], ids_v)
        pltpu.sync_copy(tone_ids_hbm.at[wid], bid_v)
        def build_fid(i, carry):
            r = i // (WI // L)
            kk = i - r * (WI // L)
            p = ids_v[r, pl.ds(kk * L, L)]
            t = bid_v[r, pl.ds(kk * L, L)]
            ids_v[r, pl.ds(kk * L, L)] = t * VP + p
            return carry
        lax.fori_loop(0, NROW * (WI // L), build_fid, 0)
        pltpu.sync_copy(bnd_ids_hbm.at[wid], bid_v)
        def build_bid(i, carry):
            r = i // (WI // L)
            kk = i - r * (WI // L)
            b = bid_v[r, pl.ds(kk * L, L)]
            bid_v[r, pl.ds(kk * L, L)] = b + MROWS
            return carry
        lax.fori_loop(0, NROW * (WI // L), build_bid, 0)

        sid = lax.axis_index("s")
        pltpu.sync_copy(ph_tab_hbm.at[pl.ds(sid * RPT, RPT)],
                        rows[0].at[pl.ds(0, RPT)])
        pltpu.sync_copy(tone_tab_hbm, rows[1].at[pl.ds(0, NT)])
        for t in range(NT):
            dt = []
            for c in range(CCH):
                v = rows[1][t, pl.ds(c * L, L)]
                if t > 0:
                    v = v - rows[1][t - 1, pl.ds(c * L, L)]
                dt.append(v)

            def add_dt(r, carry):
                for c in range(CCH):
                    plsc.addupdate(rows[0].at[r, pl.ds(c * L, L)], dt[c])
                return carry
            lax.fori_loop(0, RPT, add_dt, 0)
            pltpu.sync_copy(rows[0].at[pl.ds(0, RPT)],
                            tab_sp.at[pl.ds(t * VP + sid * RPT, RPT)])
        pltpu.sync_copy(bnd_tab_hbm, rows[1].at[pl.ds(0, NB)])
        pltpu.sync_copy(rows[1].at[pl.ds(0, NB)],
                        tab_sp.at[pl.ds(MROWS, NB)])
        plsc.subcore_barrier()

        def g_start(w, p):
            r = w // 2
            hoff = (w % 2) * W2
            pltpu.async_copy(tab_sp.at[ids_v.at[r, pl.ds(hoff, W2)]],
                             rows[p].at[pl.ds(0, W2)], gasems[p])
            pltpu.async_copy(tab_sp.at[bid_v.at[r, pl.ds(hoff, W2)]],
                             rows[p].at[pl.ds(W2, W2)], gbsems[p])

        def g_wait(w, p):
            r = w // 2
            hoff = (w % 2) * W2
            pltpu.make_async_copy(tab_sp.at[ids_v.at[r, pl.ds(hoff, W2)]],
                                  rows[p].at[pl.ds(0, W2)], gasems[p]).wait()
            pltpu.make_async_copy(tab_sp.at[bid_v.at[r, pl.ds(hoff, W2)]],
                                  rows[p].at[pl.ds(W2, W2)], gbsems[p]).wait()

        def o_copy(w, p):
            return pltpu.make_async_copy(
                rows[p].at[pl.ds(0, W2)],
                out_hbm.at[pl.ds(base + w * W2, W2)], osems[p])

        def compute(w, p):
            def pos_body(i, carry2):
                for c in range(CCH):
                    v = rows[p][W2 + i, pl.ds(c * L, L)]
                    plsc.addupdate(rows[p].at[i, pl.ds(c * L, L)], v)
                return carry2
            lax.fori_loop(0, W2, pos_body, 0)

        def step(w, par, do_owait, do_gstart):
            g_wait(w, par)
            if do_owait:
                o_copy(w - 2, (par + 1) % NBUF).wait()
            if do_gstart:
                g_start(w + 1, (par + 1) % NBUF)
            compute(w, par)
            o_copy(w, par).start()

        g_start(0, 0)

        for par in range(NBUF):
            step(par, par, par >= 2, True)

        def round_body(r, carry):
            w0 = r * NBUF
            for par in range(NBUF):
                step(w0 + par, par, True, True)
            return carry
        lax.fori_loop(1, (NWIN - 1) // NBUF, round_body, 0)

        for w in range(((NWIN - 1) // NBUF) * NBUF, NWIN):
            step(w, w % NBUF, True, w + 1 < NWIN)

        for w in (NWIN - 2, NWIN - 1):
            o_copy(w, w % NBUF).wait()

    return k


_kernel_fn = _make_kernel()


@jax.jit
def _run(ph_ids, tone_ids, boundary_ids, ph_table, tone_table, boundary_table):
    ph = ph_ids.reshape(NW, NROW, WI).astype(jnp.int32)
    tn = tone_ids.reshape(NW, NROW, WI).astype(jnp.int32)
    bd = boundary_ids.reshape(NW, NROW, WI).astype(jnp.int32)
    ph_table = jnp.concatenate(
        [ph_table, jnp.zeros((VP - ph_table.shape[0], D), ph_table.dtype)])
    out = _kernel_fn(ph, tn, bd, ph_table, tone_table, boundary_table)
    return out.reshape(B, TPH, D)


def kernel(ph_ids, tone_ids, boundary_ids, ph_table, tone_table, boundary_table):
    return _run(ph_ids, tone_ids, boundary_ids, ph_table, tone_table,
                boundary_table)


# speedup vs baseline: 1.6752x; 1.0004x over previous
import functools

import jax
import jax.numpy as jnp
from jax import lax
from jax.experimental import pallas as pl
from jax.experimental.pallas import tpu as pltpu
from jax.experimental.pallas import tpu_sc as plsc

NC, NS, L = 2, 16, 16
NW = NC * NS
D = 128
B, TPH = 1024, 200
N = B * TPH
PW = N // NW
WI = 128
NROW = PW // WI
W2 = 64
NWIN = PW // W2
NBUF = 3
NT, NB = 8, 6
CCH = D // L
VP = 1024
RPT = VP // NS
MROWS = NT * VP
REP = 16
TROWS = MROWS + NB * REP


def _make_kernel():
    mesh = plsc.VectorSubcoreMesh(core_axis_name="c", subcore_axis_name="s")

    scratch = (
        [pltpu.VMEM((NROW, WI), jnp.int32)] * 2
        + [pltpu.VMEM((2 * W2, D), jnp.float32)] * NBUF
        + [pltpu.VMEM_SHARED((TROWS, D), jnp.float32)]
        + [pltpu.SemaphoreType.DMA] * (3 * NBUF)
    )

    @functools.partial(
        pl.kernel,
        out_type=jax.ShapeDtypeStruct((N, D), jnp.float32),
        mesh=mesh,
        compiler_params=pltpu.CompilerParams(needs_layout_passes=False),
        scratch_types=scratch,
    )
    def k(ph_ids_hbm, tone_ids_hbm, bnd_ids_hbm,
          ph_tab_hbm, tone_tab_hbm, bnd_tab_hbm,
          out_hbm,
          ids_v, bid_v, *rest):
        rows = rest[:NBUF]
        tab_sp = rest[NBUF]
        gasems = rest[NBUF + 1:2 * NBUF + 1]
        gbsems = rest[2 * NBUF + 1:3 * NBUF + 1]
        osems = rest[3 * NBUF + 1:]

        wid = lax.axis_index("s") * NC + lax.axis_index("c")
        base = wid * PW

        pltpu.sync_copy(ph_ids_hbm.at[wid], ids_v)
        pltpu.sync_copy(tone_ids_hbm.at[wid], bid_v)
        def build_fid(i, carry):
            r = i // (WI // L)
            kk = i - r * (WI // L)
            p = ids_v[r, pl.ds(kk * L, L)]
            t = bid_v[r, pl.ds(kk * L, L)]
            ids_v[r, pl.ds(kk * L, L)] = t * VP + p
            return carry
        lax.fori_loop(0, NROW * (WI // L), build_fid, 0)
        pltpu.sync_copy(bnd_ids_hbm.at[wid], bid_v)
        rep_iota = lax.iota(jnp.int32, L) + MROWS
        def build_bid(i, carry):
            r = i // (WI // L)
            kk = i - r * (WI // L)
            b = bid_v[r, pl.ds(kk * L, L)]
            bid_v[r, pl.ds(kk * L, L)] = b * REP + rep_iota
            return carry
        lax.fori_loop(0, NROW * (WI // L), build_bid, 0)

        sid = lax.axis_index("s")
        pltpu.sync_copy(ph_tab_hbm.at[pl.ds(sid * RPT, RPT)],
                        rows[0].at[pl.ds(0, RPT)])
        pltpu.sync_copy(tone_tab_hbm, rows[1].at[pl.ds(0, NT)])
        for t in range(NT):
            dt = []
            for c in range(CCH):
                v = rows[1][t, pl.ds(c * L, L)]
                if t > 0:
                    v = v - rows[1][t - 1, pl.ds(c * L, L)]
                dt.append(v)

            def add_dt(r, carry):
                for c in range(CCH):
                    plsc.addupdate(rows[0].at[r, pl.ds(c * L, L)], dt[c])
                return carry
            lax.fori_loop(0, RPT, add_dt, 0)
            pltpu.sync_copy(rows[0].at[pl.ds(0, RPT)],
                            tab_sp.at[pl.ds(t * VP + sid * RPT, RPT)])
        pltpu.sync_copy(bnd_tab_hbm, rows[1].at[pl.ds(0, NB)])
        for bb in range(NB):
            for c in range(CCH):
                v = rows[1][bb, pl.ds(c * L, L)]
                for rp in range(REP):
                    rows[1][8 + bb * REP + rp, pl.ds(c * L, L)] = v
        pltpu.sync_copy(rows[1].at[pl.ds(8, NB * REP)],
                        tab_sp.at[pl.ds(MROWS, NB * REP)])
        plsc.subcore_barrier()

        def g_start(w, p):
            r = w // 2
            hoff = (w % 2) * W2
            pltpu.async_copy(tab_sp.at[ids_v.at[r, pl.ds(hoff, W2)]],
                             rows[p].at[pl.ds(0, W2)], gasems[p])
            pltpu.async_copy(tab_sp.at[bid_v.at[r, pl.ds(hoff, W2)]],
                             rows[p].at[pl.ds(W2, W2)], gbsems[p])

        def g_wait(w, p):
            r = w // 2
            hoff = (w % 2) * W2
            pltpu.make_async_copy(tab_sp.at[ids_v.at[r, pl.ds(hoff, W2)]],
                                  rows[p].at[pl.ds(0, W2)], gasems[p]).wait()
            pltpu.make_async_copy(tab_sp.at[bid_v.at[r, pl.ds(hoff, W2)]],
                                  rows[p].at[pl.ds(W2, W2)], gbsems[p]).wait()

        def o_copy(w, p):
            return pltpu.make_async_copy(
                rows[p].at[pl.ds(0, W2)],
                out_hbm.at[pl.ds(base + w * W2, W2)], osems[p])

        def compute(w, p):
            def pos_body(i, carry2):
                for c in range(CCH):
                    v = rows[p][W2 + i, pl.ds(c * L, L)]
                    plsc.addupdate(rows[p].at[i, pl.ds(c * L, L)], v)
                return carry2
            lax.fori_loop(0, W2, pos_body, 0)

        def step(w, par, do_owait, do_gstart):
            g_wait(w, par)
            if do_owait:
                o_copy(w - 2, (par + 1) % NBUF).wait()
            if do_gstart:
                g_start(w + 1, (par + 1) % NBUF)
            compute(w, par)
            o_copy(w, par).start()

        g_start(0, 0)

        for par in range(NBUF):
            step(par, par, par >= 2, True)

        def round_body(r, carry):
            w0 = r * NBUF
            for par in range(NBUF):
                step(w0 + par, par, True, True)
            return carry
        lax.fori_loop(1, (NWIN - 1) // NBUF, round_body, 0)

        for w in range(((NWIN - 1) // NBUF) * NBUF, NWIN):
            step(w, w % NBUF, True, w + 1 < NWIN)

        for w in (NWIN - 2, NWIN - 1):
            o_copy(w, w % NBUF).wait()

    return k


_kernel_fn = _make_kernel()


@jax.jit
def _run(ph_ids, tone_ids, boundary_ids, ph_table, tone_table, boundary_table):
    ph = ph_ids.reshape(NW, NROW, WI).astype(jnp.int32)
    tn = tone_ids.reshape(NW, NROW, WI).astype(jnp.int32)
    bd = boundary_ids.reshape(NW, NROW, WI).astype(jnp.int32)
    ph_table = jnp.concatenate(
        [ph_table, jnp.zeros((VP - ph_table.shape[0], D), ph_table.dtype)])
    out = _kernel_fn(ph, tn, bd, ph_table, tone_table, boundary_table)
    return out.reshape(B, TPH, D)


def kernel(ph_ids, tone_ids, boundary_ids, ph_table, tone_table, boundary_table):
    return _run(ph_ids, tone_ids, boundary_ids, ph_table, tone_table,
                boundary_table)
